# Initial kernel scaffold; baseline (speedup 1.0000x reference)
import jax
import jax.numpy as jnp
from jax.experimental import pallas as pl


def kernel(z, embedding):
    raise NotImplementedError("write your pallas kernel here")


# speedup vs baseline: 1.1681x; 1.1681x over previous
import jax
import jax.numpy as jnp
from jax import lax
from jax.experimental import pallas as pl
from jax.experimental.pallas import tpu as pltpu

_K = 8192
_C = 64
_BETA = 0.25
_B, _H, _W = 8, 32, 32
_HW = _H * _W
_KCHUNK = 512
_NCHUNKS = _K // _KCHUNK


def _vq_body(z_ref, e_ref, ids_ref, zq_ref, loss_ref, acc_ref):
    b = pl.program_id(0)
    z = z_ref[0]
    nsq = jnp.sum(z * z, axis=0, keepdims=True)
    n = jnp.sqrt(nsq)
    zn = z / jnp.maximum(n, 1e-12)
    zn_sq = jnp.sum(zn * zn)

    run_max = jnp.full((1, _HW), -jnp.inf, dtype=jnp.float32)
    run_idx = jnp.zeros((1, _HW), dtype=jnp.int32)

    def pass1(k, carry):
        rmax, ridx = carry
        e_blk = e_ref[pl.ds(k * _KCHUNK, _KCHUNK), :]
        sim = lax.dot(e_blk, zn)
        bmax = jnp.max(sim, axis=0, keepdims=True)
        iota = lax.broadcasted_iota(jnp.int32, sim.shape, 0)
        bidx = jnp.min(jnp.where(sim == bmax, iota, _K),
                       axis=0, keepdims=True) + k * _KCHUNK
        better = bmax > rmax
        return (jnp.where(better, bmax, rmax),
                jnp.where(better, bidx, ridx))

    run_max, run_idx = lax.fori_loop(0, _NCHUNKS, pass1, (run_max, run_idx))
    ids_ref[0] = run_idx

    acc_ref[...] = jnp.zeros((_C, _HW), jnp.float32)

    def pass2(k, _):
        e_blk = e_ref[pl.ds(k * _KCHUNK, _KCHUNK), :]
        iota = lax.broadcasted_iota(jnp.int32, (_KCHUNK, _HW), 0) + k * _KCHUNK
        onehot = (iota == run_idx).astype(jnp.float32)
        acc_ref[...] += lax.dot_general(
            e_blk, onehot, (((0,), (0,)), ((), ())))
        return 0

    lax.fori_loop(0, _NCHUNKS, pass2, 0)
    zq = acc_ref[...]
    zq_ref[0] = zq

    batch_term = jnp.sum(zq * zq) - 2.0 * jnp.sum(run_max) + zn_sq

    @pl.when(b == 0)
    def _():
        loss_ref[...] = jnp.zeros((1, 1), jnp.float32)

    loss_ref[...] += jnp.full((1, 1), (_BETA / (_B * _HW * _C)), jnp.float32) * batch_term


def kernel(z, embedding):
    zf = z.reshape(_B, _C, _HW)
    ids3, zq3, loss = pl.pallas_call(
        _vq_body,
        grid=(_B,),
        in_specs=[
            pl.BlockSpec((1, _C, _HW), lambda b: (b, 0, 0)),
            pl.BlockSpec((_K, _C), lambda b: (0, 0)),
        ],
        out_specs=[
            pl.BlockSpec((1, 1, _HW), lambda b: (b, 0, 0)),
            pl.BlockSpec((1, _C, _HW), lambda b: (b, 0, 0)),
            pl.BlockSpec((1, 1), lambda b: (0, 0)),
        ],
        out_shape=[
            jax.ShapeDtypeStruct((_B, 1, _HW), jnp.int32),
            jax.ShapeDtypeStruct((_B, _C, _HW), jnp.float32),
            jax.ShapeDtypeStruct((1, 1), jnp.float32),
        ],
        scratch_shapes=[pltpu.VMEM((_C, _HW), jnp.float32)],
    )(zf, embedding)
    z_q_out = zq3.reshape(_B, _C, _H, _W)
    token_ids = ids3.reshape(_B, _H, _W)
    return (z_q_out, loss[0, 0], token_ids)


# speedup vs baseline: 1.4969x; 1.2815x over previous
import jax
import jax.numpy as jnp
from jax import lax
from jax.experimental import pallas as pl
from jax.experimental.pallas import tpu as pltpu

_K = 8192
_C = 64
_BETA = 0.25
_B, _H, _W = 8, 32, 32
_HW = _H * _W
_KC = 512
_NCH = _K // _KC


def _vq_body(z_ref, e_ref, et_ref, ids_ref, zq_ref, loss_ref, s_ref, acc_ref):
    b = pl.program_id(0)
    z = z_ref[0]
    nsq = jnp.sum(z * z, axis=0, keepdims=True)
    n = jnp.sqrt(nsq)
    zn = z / jnp.maximum(n, 1e-12)
    zn_sq = jnp.sum(zn * zn)

    def pass1(k, rmax):
        e_blk = e_ref[pl.ds(k * _KC, _KC), :]
        sim = lax.dot(e_blk, zn)
        s_ref[k] = sim
        return jnp.maximum(rmax, jnp.max(sim, axis=0, keepdims=True))

    rmax = lax.fori_loop(
        0, _NCH, pass1, jnp.full((1, _HW), -jnp.inf, dtype=jnp.float32))

    acc_ref[...] = jnp.zeros((_C + 3, _HW), jnp.float32)

    def pass2(k, _):
        onehot = (s_ref[k] == rmax).astype(jnp.float32)
        g_blk = et_ref[:, pl.ds(k * _KC, _KC)]
        acc_ref[...] += lax.dot(g_blk, onehot)
        return 0

    lax.fori_loop(0, _NCH, pass2, 0)
    acc = acc_ref[...]
    count = acc[_C + 2:_C + 3]
    ids = (acc[_C:_C + 1] * 128.0 + acc[_C + 1:_C + 2]).astype(jnp.int32)
    ids_ref[0] = ids
    zq_ref[0] = acc[:_C]

    has_tie = jnp.any(count != 1.0)

    @pl.when(has_tie)
    def _():
        def find(k, ridx):
            iota = lax.broadcasted_iota(jnp.int32, (_KC, _HW), 0) + k * _KC
            bidx = jnp.min(jnp.where(s_ref[k] == rmax, iota, _K),
                           axis=0, keepdims=True)
            return jnp.minimum(ridx, bidx)

        ids_x = lax.fori_loop(0, _NCH, find,
                              jnp.full((1, _HW), _K, dtype=jnp.int32))
        ids_ref[0] = ids_x
        acc_ref[...] = jnp.zeros((_C + 3, _HW), jnp.float32)

        def rebuild(k, _):
            iota = lax.broadcasted_iota(jnp.int32, (_KC, _HW), 0) + k * _KC
            onehot = (iota == ids_x).astype(jnp.float32)
            g_blk = et_ref[:, pl.ds(k * _KC, _KC)]
            acc_ref[...] += lax.dot(g_blk, onehot)
            return 0

        lax.fori_loop(0, _NCH, rebuild, 0)
        zq_ref[0] = acc_ref[:_C]

    zq = zq_ref[0]
    batch_term = jnp.sum(zq * zq) - 2.0 * jnp.sum(rmax) + zn_sq

    @pl.when(b == 0)
    def _():
        loss_ref[...] = jnp.zeros((1, 1), jnp.float32)

    loss_ref[...] += jnp.full((1, 1), (_BETA / (_B * _HW * _C)),
                              jnp.float32) * batch_term


def kernel(z, embedding):
    zf = z.reshape(_B, _C, _HW)
    kio = jnp.arange(_K, dtype=jnp.float32)
    et_aug = jnp.concatenate(
        [embedding.T,
         jnp.floor(kio / 128.0)[None, :],
         jnp.mod(kio, 128.0)[None, :],
         jnp.ones((1, _K), jnp.float32)], axis=0)
    ids3, zq3, loss = pl.pallas_call(
        _vq_body,
        grid=(_B,),
        in_specs=[
            pl.BlockSpec((1, _C, _HW), lambda b: (b, 0, 0)),
            pl.BlockSpec((_K, _C), lambda b: (0, 0)),
            pl.BlockSpec((_C + 3, _K), lambda b: (0, 0)),
        ],
        out_specs=[
            pl.BlockSpec((1, 1, _HW), lambda b: (b, 0, 0)),
            pl.BlockSpec((1, _C, _HW), lambda b: (b, 0, 0)),
            pl.BlockSpec((1, 1), lambda b: (0, 0)),
        ],
        out_shape=[
            jax.ShapeDtypeStruct((_B, 1, _HW), jnp.int32),
            jax.ShapeDtypeStruct((_B, _C, _HW), jnp.float32),
            jax.ShapeDtypeStruct((1, 1), jnp.float32),
        ],
        scratch_shapes=[
            pltpu.VMEM((_NCH, _KC, _HW), jnp.float32),
            pltpu.VMEM((_C + 3, _HW), jnp.float32),
        ],
    )(zf, embedding, et_aug)
    z_q_out = zq3.reshape(_B, _C, _H, _W)
    token_ids = ids3.reshape(_B, _H, _W)
    return (z_q_out, loss[0, 0], token_ids)
